# baseline (speedup 1.0000x reference)
import jax
import jax.numpy as jnp
from jax.experimental import pallas as pl

N = 512
E = 8192
K = 4
NORB = 4


def kernel(edge_features, node_features, atom_types, kpoints, edge_index, edge_cell_shift):
    def body(x_ref, o_ref):
        o_ref[...] = x_ref[...]

    edge_features = pl.pallas_call(
        body,
        out_shape=jax.ShapeDtypeStruct(edge_features.shape, edge_features.dtype),
    )(edge_features)

    hop = jnp.zeros((E, NORB, NORB), dtype=edge_features.dtype)
    hop = hop.at[:, 0:1, 0:1].set(edge_features[:, 0:1].reshape(-1, 1, 1))
    hop = hop.at[:, 0:1, 1:4].set(edge_features[:, 1:4].reshape(-1, 1, 3))
    hop = hop.at[:, 1:4, 0:1].set(edge_features[:, 4:7].reshape(-1, 3, 1))
    hop = hop.at[:, 1:4, 1:4].set(edge_features[:, 7:16].reshape(-1, 3, 3))
    ons = jnp.zeros((N, NORB, NORB), dtype=node_features.dtype)
    ons = ons.at[:, 0:1, 0:1].set(node_features[:, 0:1].reshape(-1, 1, 1))
    ons = ons.at[:, 0:1, 1:4].set(node_features[:, 1:4].reshape(-1, 1, 3))
    ons = ons.at[:, 1:4, 0:1].set(jnp.transpose(node_features[:, 1:4].reshape(-1, 1, 3), (0, 2, 1)))
    ons = ons.at[:, 1:4, 1:4].set(node_features[:, 4:13].reshape(-1, 3, 3))
    ii = edge_index[0]
    jj = edge_index[1]
    phase = jnp.exp(-1j * 2.0 * jnp.pi * (kpoints @ edge_cell_shift.T)).astype(jnp.complex64)
    contrib = hop[None, :, :, :].astype(jnp.complex64) * phase[:, :, None, None]
    blk = jnp.zeros((K, N, N, NORB, NORB), dtype=jnp.complex64)
    diag = jnp.arange(N)
    blk = blk.at[:, diag, diag].add((0.5 * ons.astype(jnp.complex64))[None, :, :, :])
    blk = blk.at[:, ii, jj].add(contrib)
    block = blk.transpose(0, 1, 3, 2, 4).reshape(K, N * NORB, N * NORB)
    block = block + jnp.conj(jnp.swapaxes(block, 1, 2))
    return block


# speedup vs baseline: 5.5058x; 5.5058x over previous
import functools

import jax
import jax.numpy as jnp
from jax import lax
from jax.experimental import pallas as pl
from jax.experimental.pallas import tpu as pltpu
from jax.experimental.pallas import tpu_sc as plsc

N = 512
E = 8192
K = 4
NORB = 4
S = 2 * E
EB = 512
CHUNK = 64
GPC = CHUNK // 16
NTILES = 32
APT = N // NTILES

_F = [0, 1, 2, 3, 4, 7, 8, 9, 5, 10, 11, 12, 6, 13, 14, 15]
_FT = [_F[4 * (m % 4) + m // 4] for m in range(16)]
_A = [0, 1, 2, 3, 1, 4, 5, 6, 2, 7, 8, 9, 3, 10, 11, 12]
_AT = [_A[4 * (m % 4) + m // 4] for m in range(16)]


def _edge_table_body(kp_ref, ef_ref, sh_ref, o_ref):
    k = pl.program_id(0)
    side = pl.program_id(1)
    kx, ky, kz = kp_ref[k, 0], kp_ref[k, 1], kp_ref[k, 2]
    def _bf(x):
        return x.astype(jnp.bfloat16).astype(jnp.float32)

    d = (
        _bf(kx) * _bf(sh_ref[:, 0:1])
        + _bf(ky) * _bf(sh_ref[:, 1:2])
        + _bf(kz) * _bf(sh_ref[:, 2:3])
    )
    theta = (2.0 * jnp.pi) * (d - jnp.round(d))
    cth = jnp.cos(theta)
    sth = jnp.sin(theta)
    hop_d = jnp.concatenate([ef_ref[:, f : f + 1] for f in _F], axis=1)
    hop_t = jnp.concatenate([ef_ref[:, f : f + 1] for f in _FT], axis=1)
    hsel = jnp.where(side == 0, hop_d, hop_t)
    sgn = jnp.where(side == 0, -1.0, 1.0)
    re = hsel * cth
    im = (sgn * hsel) * sth
    pad = jnp.zeros((re.shape[0], 96), jnp.float32)
    o_ref[0] = jnp.concatenate([re, im, pad], axis=1)


def _onsite_table_body(nf_ref, o_ref):
    ca = jnp.concatenate([nf_ref[:, f : f + 1] for f in _A], axis=1)
    cb = jnp.concatenate([nf_ref[:, f : f + 1] for f in _AT], axis=1)
    re = 0.5 * (ca + cb)
    pad = jnp.zeros((re.shape[0], 112), jnp.float32)
    o_ref[...] = jnp.concatenate([re, pad], axis=1)


def _sc_assemble_body(t_ref, td_ref, sid_ref, col_ref, off_ref, zer_ref,
                      out_ref, sid_v, col_v, off_v, idx_v, stage_v, dst_v,
                      buf_v, sem):
    wid = lax.axis_index("s") * 2 + lax.axis_index("c")
    pltpu.sync_copy(sid_ref, sid_v)
    pltpu.sync_copy(col_ref, col_v)
    pltpu.sync_copy(off_ref, off_v)
    iota = lax.iota(jnp.int32, 16)
    va = off_v[pl.ds(wid * 16, 16)]
    vb = off_v[pl.ds(wid * 16 + 16, 16)]

    def _lane(v, a):
        eq = 1 - jnp.minimum(jnp.abs(iota - a), 1)
        return jnp.max(v * eq)

    def atom_body(a, _):
        i = wid * APT + a
        off_i = _lane(va, a)
        off_n = jnp.where(a == APT - 1, _lane(vb, 0), _lane(va, a + 1))
        g0 = off_i // 16
        g1 = (off_n + 15) // 16
        nch = (g1 - g0 + GPC - 1) // GPC
        pltpu.sync_copy(td_ref.at[i], dst_v)
        for k in range(K):
            pltpu.sync_copy(zer_ref, buf_v.at[pl.ds(0, 8)])

            def chunk_body(q, _, k=k):
                base = (g0 + q * GPC) * 16
                for g in range(GPC):
                    sid16 = sid_v[pl.ds(base + 16 * g, 16)]
                    idx_v[pl.ds(16 * g, 16)] = sid16 + (k * S)
                pltpu.async_copy(t_ref.at[idx_v], stage_v, sem).wait()
                for g in range(GPC):
                    pos = base + 16 * g + iota
                    valid = (pos >= off_i) & (pos < off_n)
                    col16 = col_v[pl.ds(base + 16 * g, 16)]
                    col4 = col16 * 4
                    svec = iota + 16 * g
                    trash = jnp.full((16,), 8, jnp.int32)
                    for t in range(32):
                        mvec = (iota + t) & 31
                        ent = mvec & 15
                        plane = mvec >> 4
                        srow = ((ent >> 2) << 1) + plane
                        val = plsc.load_gather(stage_v, [svec, mvec])
                        rowv = jnp.where(valid, srow, trash)
                        colv = col4 + (ent & 3)
                        plsc.addupdate_scatter(buf_v, [rowv, colv], val)
                return _

            lax.fori_loop(0, nch, chunk_body, 0)
            dre = dst_v[pl.ds(0, 16)]
            drow = (iota // 4) * 2
            dcol = (iota % 4) + i * 4
            plsc.addupdate_scatter(buf_v, [drow, dcol], dre)
            pltpu.sync_copy(
                buf_v.at[pl.ds(0, 8)],
                out_ref.at[pl.ds((k * N) * 8 + i * 8, 8)],
            )
        return _

    lax.fori_loop(0, APT, atom_body, 0)


def kernel(edge_features, node_features, atom_types, kpoints, edge_index,
           edge_cell_shift):
    tbl = pl.pallas_call(
        _edge_table_body,
        grid=(K, 2, E // EB),
        in_specs=[
            pl.BlockSpec((K, 3), lambda k, s, b: (0, 0),
                         memory_space=pltpu.SMEM),
            pl.BlockSpec((EB, 16), lambda k, s, b: (b, 0)),
            pl.BlockSpec((EB, 3), lambda k, s, b: (b, 0)),
        ],
        out_specs=pl.BlockSpec(
            (1, EB, 128), lambda k, s, b: (k, s * (E // EB) + b, 0)
        ),
        out_shape=jax.ShapeDtypeStruct((K, S, 128), jnp.float32),
    )(kpoints, edge_features, edge_cell_shift)
    tbl = tbl.reshape(K * S, 128)

    tdiag = pl.pallas_call(
        _onsite_table_body,
        out_shape=jax.ShapeDtypeStruct((N, 128), jnp.float32),
    )(node_features)

    ii = edge_index[0]
    jj = edge_index[1]
    tgt = jnp.concatenate([ii, jj]).astype(jnp.int32)
    colv = jnp.concatenate([jj, ii]).astype(jnp.int32)
    order = jnp.argsort(tgt)
    sorted_sid = order.astype(jnp.int32)
    sorted_col = jnp.take(colv, order).astype(jnp.int32)
    sorted_tgt = jnp.take(tgt, order)
    off = jnp.searchsorted(
        sorted_tgt, jnp.arange(N + 1, dtype=jnp.int32)
    ).astype(jnp.int32)
    pad = jnp.zeros((128,), jnp.int32)
    sorted_sid = jnp.concatenate([sorted_sid, pad])
    sorted_col = jnp.concatenate([sorted_col, pad])
    off = jnp.concatenate([off, jnp.zeros((127,), jnp.int32)])
    zeros = jnp.zeros((8, 2048), jnp.float32)

    mesh = plsc.VectorSubcoreMesh(core_axis_name="c", subcore_axis_name="s")
    sc = functools.partial(
        pl.kernel,
        out_type=jax.ShapeDtypeStruct((K * N * 8, 2048), jnp.float32),
        mesh=mesh,
        compiler_params=pltpu.CompilerParams(needs_layout_passes=False),
        scratch_types=[
            pltpu.VMEM((S + 128,), jnp.int32),
            pltpu.VMEM((S + 128,), jnp.int32),
            pltpu.VMEM((640,), jnp.int32),
            pltpu.VMEM((CHUNK,), jnp.int32),
            pltpu.VMEM((CHUNK, 128), jnp.float32),
            pltpu.VMEM((128,), jnp.float32),
            pltpu.VMEM((9, 2048), jnp.float32),
            pltpu.SemaphoreType.DMA,
        ],
    )(_sc_assemble_body)
    dense = sc(tbl, tdiag, sorted_sid, sorted_col, off, zeros)

    x = dense.reshape(K, N * NORB, 2, N * NORB)
    return lax.complex(x[:, :, 0, :], x[:, :, 1, :])
